# speedup vs baseline: 2.0751x; 2.0751x over previous
import functools

import jax
import jax.numpy as jnp
from jax.experimental import pallas as pl
from jax.experimental.pallas import tpu as pltpu

K_TOP_ = 64
S_LEN = 8192
C_BLK = 128


def _topk_body(in_ref, out_ref):
    x = in_ref[0]
    rows = jax.lax.broadcasted_iota(jnp.int32, (S_LEN, C_BLK), 0)

    def step(j, x):
        m = jnp.max(x, axis=0)
        am = jnp.argmax(x, axis=0)
        out_ref[0, j, :] = m
        return jnp.where(rows == am[None, :], -jnp.inf, x)

    jax.lax.fori_loop(0, K_TOP_, step, x)


@jax.jit
def kernel(inputs):
    b, s, c = inputs.shape
    grid = (b, c // C_BLK)
    return pl.pallas_call(
        _topk_body,
        grid=grid,
        in_specs=[pl.BlockSpec((1, s, C_BLK), lambda i, j: (i, 0, j))],
        out_specs=pl.BlockSpec((1, K_TOP_, C_BLK), lambda i, j: (i, 0, j)),
        out_shape=jax.ShapeDtypeStruct((b, K_TOP_, c), jnp.float32),
    )(inputs)


# speedup vs baseline: 11.2979x; 5.4445x over previous
import functools

import jax
import jax.numpy as jnp
from jax import lax
from jax.experimental import pallas as pl
from jax.experimental.pallas import tpu as pltpu
from jax.experimental.pallas import tpu_sc as plsc

K_TOP_ = 64
B_ = 4
S_ = 8192
C_ = 1024
L_ = 16
NW_ = 32
CBLK_ = C_ // L_
TASKS_ = B_ * CBLK_
TPW_ = TASKS_ // NW_
CH_ = 2048
NCHUNK_ = S_ // CH_
BK_ = 8
CAP_ = 32
TRIG_ = CAP_ - BK_
NEG_ = float("-inf")


def _sort32_asc(v):
    n = 32
    k = 2
    while k <= n:
        j = k // 2
        while j >= 1:
            for i in range(n):
                ix = i ^ j
                if ix > i:
                    a, b = v[i], v[ix]
                    lo = jnp.minimum(a, b)
                    hi = jnp.maximum(a, b)
                    if (i & k) == 0:
                        v[i], v[ix] = lo, hi
                    else:
                        v[i], v[ix] = hi, lo
            j //= 2
        k *= 2


def _bmerge32_desc(v):
    for d in (16, 8, 4, 2, 1):
        for i in range(32):
            if (i % (2 * d)) < d:
                a, b = v[i], v[i + d]
                v[i] = jnp.maximum(a, b)
                v[i + d] = jnp.minimum(a, b)


def _make_kernel():
    mesh = plsc.VectorSubcoreMesh(core_axis_name="c", subcore_axis_name="s")

    @functools.partial(
        pl.kernel,
        mesh=mesh,
        compiler_params=pltpu.CompilerParams(
            use_tc_tiling_on_sc=False, needs_layout_passes=False
        ),
        out_type=jax.ShapeDtypeStruct((B_, K_TOP_, C_), jnp.float32),
        scratch_types=[
            pltpu.VMEM((2, CH_, L_), jnp.float32),
            pltpu.VMEM((K_TOP_, L_), jnp.float32),
            pltpu.VMEM((CAP_, L_), jnp.float32),
            pltpu.SemaphoreType.DMA((2,)),
        ],
    )
    def sc_topk(in_hbm, out_hbm, chunks, top, cand, sems):
        wid = lax.axis_index("s") * 2 + lax.axis_index("c")
        lanes = lax.iota(jnp.int32, 16)
        ninf16 = jnp.full((L_,), NEG_, jnp.float32)
        zero16 = jnp.zeros((L_,), jnp.int32)

        def fold(t, cnt):
            c = [cand[i] for i in range(CAP_)]
            _sort32_asc(c)
            for jj in range(32):
                top[32 + jj] = jnp.maximum(top[32 + jj], c[jj])
            up = [None] * 32
            lo = [None] * 32
            for i in range(32):
                a = top[i]
                b = top[32 + i]
                up[i] = jnp.maximum(a, b)
                lo[i] = jnp.minimum(a, b)
            _bmerge32_desc(up)
            for i in range(32):
                top[i] = up[i]
            _bmerge32_desc(lo)
            for i in range(32):
                top[32 + i] = lo[i]
                cand[i] = ninf16
            return top[63], zero16

        def passthru(t, cnt):
            return t, cnt

        def run_task(ti, carry0):
            task = wid * TPW_ + ti
            b = task // CBLK_
            c0 = (task % CBLK_) * L_
            for r in range(K_TOP_):
                top[r] = ninf16
            for r in range(CAP_):
                cand[r] = ninf16
            pltpu.make_async_copy(
                in_hbm.at[b, pl.ds(0, CH_), pl.ds(c0, L_)],
                chunks.at[0],
                sems.at[0],
            ).start()

            def run_chunk(ch, carry):
                t, cnt = carry
                slot = lax.rem(ch, 2)
                pltpu.make_async_copy(
                    in_hbm.at[b, pl.ds(ch * CH_, CH_), pl.ds(c0, L_)],
                    chunks.at[slot],
                    sems.at[slot],
                ).wait()

                @pl.when(ch + 1 < NCHUNK_)
                def _():
                    nslot = lax.rem(ch + 1, 2)
                    pltpu.make_async_copy(
                        in_hbm.at[b, pl.ds((ch + 1) * CH_, CH_), pl.ds(c0, L_)],
                        chunks.at[nslot],
                        sems.at[nslot],
                    ).start()

                def run_blk(blk, carry2):
                    t2, cnt2 = carry2
                    base = blk * BK_
                    for r in range(BK_):
                        x = chunks[slot, base + r]
                        m = x > t2
                        plsc.store_scatter(cand, [cnt2, lanes], x, mask=m)
                        cnt2 = cnt2 + m.astype(jnp.int32)
                    mx = jnp.max(cnt2)
                    return lax.cond(mx > TRIG_, fold, passthru, t2, cnt2)

                return lax.fori_loop(0, CH_ // BK_, run_blk, (t, cnt))

            t, cnt = lax.fori_loop(0, NCHUNK_, run_chunk, (ninf16, zero16))
            t, cnt = fold(t, cnt)
            pltpu.sync_copy(top, out_hbm.at[b, pl.ds(0, K_TOP_), pl.ds(c0, L_)])
            return carry0

        lax.fori_loop(0, TPW_, run_task, 0)

    return sc_topk


_SC_TOPK = _make_kernel()


@jax.jit
def kernel(inputs):
    return _SC_TOPK(inputs)


# speedup vs baseline: 13.0767x; 1.1575x over previous
import functools

import jax
import jax.numpy as jnp
from jax import lax
from jax.experimental import pallas as pl
from jax.experimental.pallas import tpu as pltpu
from jax.experimental.pallas import tpu_sc as plsc

K_TOP_ = 64
B_ = 4
S_ = 8192
C_ = 1024
L_ = 16
NW_ = 32
SB_ = 128
NSB_ = C_ // SB_
NG_ = SB_ // L_
CH_ = 256
NCHUNK_ = S_ // CH_
BK_ = 8
CAP_ = 32
TRIG_ = CAP_ - BK_
NEG_ = float("-inf")


def _sort32_asc(v):
    n = 32
    k = 2
    while k <= n:
        j = k // 2
        while j >= 1:
            for i in range(n):
                ix = i ^ j
                if ix > i:
                    a, b = v[i], v[ix]
                    lo = jnp.minimum(a, b)
                    hi = jnp.maximum(a, b)
                    if (i & k) == 0:
                        v[i], v[ix] = lo, hi
                    else:
                        v[i], v[ix] = hi, lo
            j //= 2
        k *= 2


def _bmerge32_desc(v):
    for d in (16, 8, 4, 2, 1):
        for i in range(32):
            if (i % (2 * d)) < d:
                a, b = v[i], v[i + d]
                v[i] = jnp.maximum(a, b)
                v[i + d] = jnp.minimum(a, b)


def _make_kernel():
    mesh = plsc.VectorSubcoreMesh(core_axis_name="c", subcore_axis_name="s")

    @functools.partial(
        pl.kernel,
        mesh=mesh,
        compiler_params=pltpu.CompilerParams(needs_layout_passes=False),
        out_type=jax.ShapeDtypeStruct((B_, K_TOP_, C_), jnp.float32),
        scratch_types=[
            pltpu.VMEM((2, CH_, SB_), jnp.float32),
            pltpu.VMEM((K_TOP_, SB_), jnp.float32),
            pltpu.VMEM((CAP_, SB_), jnp.float32),
            pltpu.VMEM((NG_, L_), jnp.float32),
            pltpu.VMEM((NG_, L_), jnp.int32),
            pltpu.SemaphoreType.DMA((2,)),
        ],
    )
    def sc_topk(in_hbm, out_hbm, chunks, top, cand, thr, cnts, sems):
        wid = lax.axis_index("s") * 2 + lax.axis_index("c")
        b = wid // NSB_
        cbase = (wid % NSB_) * SB_
        lanes = lax.iota(jnp.int32, 16)
        ninf16 = jnp.full((L_,), NEG_, jnp.float32)
        zero16 = jnp.zeros((L_,), jnp.int32)

        def fold(g, t, cnt):
            gc = g * L_
            c = [cand[i, pl.ds(gc, L_)] for i in range(CAP_)]
            _sort32_asc(c)
            for jj in range(32):
                top[32 + jj, pl.ds(gc, L_)] = jnp.maximum(
                    top[32 + jj, pl.ds(gc, L_)], c[jj]
                )
            up = [None] * 32
            lo = [None] * 32
            for i in range(32):
                a = top[i, pl.ds(gc, L_)]
                bb = top[32 + i, pl.ds(gc, L_)]
                up[i] = jnp.maximum(a, bb)
                lo[i] = jnp.minimum(a, bb)
            _bmerge32_desc(up)
            for i in range(32):
                top[i, pl.ds(gc, L_)] = up[i]
            _bmerge32_desc(lo)
            for i in range(32):
                top[32 + i, pl.ds(gc, L_)] = lo[i]
                cand[i, pl.ds(gc, L_)] = ninf16
            return top[63, pl.ds(gc, L_)], zero16

        def passthru(g, t, cnt):
            return t, cnt

        def init_group(g, carry):
            gc = g * L_
            for r in range(K_TOP_):
                top[r, pl.ds(gc, L_)] = ninf16
            for r in range(CAP_):
                cand[r, pl.ds(gc, L_)] = ninf16
            thr[g] = ninf16
            cnts[g] = zero16
            return carry

        lax.fori_loop(0, NG_, init_group, 0)

        pltpu.make_async_copy(
            in_hbm.at[b, pl.ds(0, CH_), pl.ds(cbase, SB_)],
            chunks.at[0],
            sems.at[0],
        ).start()

        def run_chunk(ch, carry):
            slot = lax.rem(ch, 2)
            pltpu.make_async_copy(
                in_hbm.at[b, pl.ds(ch * CH_, CH_), pl.ds(cbase, SB_)],
                chunks.at[slot],
                sems.at[slot],
            ).wait()

            @pl.when(ch + 1 < NCHUNK_)
            def _():
                nslot = lax.rem(ch + 1, 2)
                pltpu.make_async_copy(
                    in_hbm.at[b, pl.ds((ch + 1) * CH_, CH_), pl.ds(cbase, SB_)],
                    chunks.at[nslot],
                    sems.at[nslot],
                ).start()

            def run_group(g, carry1):
                gc = g * L_

                def run_blk(blk, carry2):
                    t2, cnt2 = carry2
                    base = blk * BK_
                    cols = lanes + gc
                    for r in range(BK_):
                        x = chunks[slot, base + r, pl.ds(gc, L_)]
                        m = x > t2
                        plsc.store_scatter(cand, [cnt2, cols], x, mask=m)
                        cnt2 = cnt2 + m.astype(jnp.int32)
                    return lax.cond(
                        jnp.any(cnt2 > TRIG_), fold, passthru, g, t2, cnt2
                    )

                t, cnt = lax.fori_loop(
                    0, CH_ // BK_, run_blk, (thr[g], cnts[g])
                )
                thr[g] = t
                cnts[g] = cnt
                return carry1

            lax.fori_loop(0, NG_, run_group, 0)
            return carry

        lax.fori_loop(0, NCHUNK_, run_chunk, 0)

        def finish_group(g, carry):
            fold(g, thr[g], cnts[g])
            return carry

        lax.fori_loop(0, NG_, finish_group, 0)
        pltpu.sync_copy(top, out_hbm.at[b, pl.ds(0, K_TOP_), pl.ds(cbase, SB_)])

    return sc_topk


_SC_TOPK = _make_kernel()


@jax.jit
def kernel(inputs):
    return _SC_TOPK(inputs)


# speedup vs baseline: 16.0269x; 1.2256x over previous
import functools

import jax
import jax.numpy as jnp
from jax import lax
from jax.experimental import pallas as pl
from jax.experimental.pallas import tpu as pltpu
from jax.experimental.pallas import tpu_sc as plsc

K_TOP_ = 64
B_ = 4
S_ = 8192
C_ = 1024
L_ = 16
NW_ = 32
SB_ = 128
NSB_ = C_ // SB_
NG_ = SB_ // L_
CH_ = 256
NCHUNK_ = S_ // CH_
WIN_ = 64
NSLAB_ = 3
CAP_ = 32 * NSLAB_
NEG_ = float("-inf")


def _sort32_asc(v):
    n = 32
    k = 2
    while k <= n:
        j = k // 2
        while j >= 1:
            for i in range(n):
                ix = i ^ j
                if ix > i:
                    a, b = v[i], v[ix]
                    lo = jnp.minimum(a, b)
                    hi = jnp.maximum(a, b)
                    if (i & k) == 0:
                        v[i], v[ix] = lo, hi
                    else:
                        v[i], v[ix] = hi, lo
            j //= 2
        k *= 2


def _bmerge32_desc(v):
    for d in (16, 8, 4, 2, 1):
        for i in range(32):
            if (i % (2 * d)) < d:
                a, b = v[i], v[i + d]
                v[i] = jnp.maximum(a, b)
                v[i + d] = jnp.minimum(a, b)


def _make_kernel():
    mesh = plsc.VectorSubcoreMesh(core_axis_name="c", subcore_axis_name="s")

    @functools.partial(
        pl.kernel,
        mesh=mesh,
        compiler_params=pltpu.CompilerParams(needs_layout_passes=False),
        out_type=jax.ShapeDtypeStruct((B_, K_TOP_, C_), jnp.float32),
        scratch_types=[
            pltpu.VMEM((2, CH_, SB_), jnp.float32),
            pltpu.VMEM((K_TOP_, SB_), jnp.float32),
            pltpu.VMEM((CAP_, SB_), jnp.float32),
            pltpu.VMEM((NG_, L_), jnp.float32),
            pltpu.VMEM((NG_, L_), jnp.int32),
            pltpu.SemaphoreType.DMA((2,)),
        ],
    )
    def sc_topk(in_hbm, out_hbm, chunks, top, cand, thr, cnts, sems):
        wid = lax.axis_index("s") * 2 + lax.axis_index("c")
        b = wid // NSB_
        cbase = (wid % NSB_) * SB_
        lanes = lax.iota(jnp.int32, 16)
        ninf16 = jnp.full((L_,), NEG_, jnp.float32)
        zero16 = jnp.zeros((L_,), jnp.int32)

        def fold(g, s, t, cnt):
            gc = g * L_
            sb = s * 32
            c = [cand[sb + i, pl.ds(gc, L_)] for i in range(32)]
            _sort32_asc(c)
            for jj in range(32):
                top[32 + jj, pl.ds(gc, L_)] = jnp.maximum(
                    top[32 + jj, pl.ds(gc, L_)], c[jj]
                )
            up = [None] * 32
            lo = [None] * 32
            for i in range(32):
                a = top[i, pl.ds(gc, L_)]
                bb = top[32 + i, pl.ds(gc, L_)]
                up[i] = jnp.maximum(a, bb)
                lo[i] = jnp.minimum(a, bb)
            _bmerge32_desc(up)
            for i in range(32):
                top[i, pl.ds(gc, L_)] = up[i]
            _bmerge32_desc(lo)
            for i in range(32):
                top[32 + i, pl.ds(gc, L_)] = lo[i]
                cand[sb + i, pl.ds(gc, L_)] = ninf16
            return top[63, pl.ds(gc, L_)], zero16

        def fold_all(g, t, cnt):
            mx = jnp.max(cnt)
            nslab = (mx + 31) // 32

            def one(s, tc):
                return fold(g, s, *tc)

            return lax.fori_loop(0, nslab, one, (t, cnt))

        def passthru(g, t, cnt):
            return t, cnt

        def init_group(g, carry):
            gc = g * L_
            for r in range(K_TOP_):
                top[r, pl.ds(gc, L_)] = ninf16
            for r in range(CAP_):
                cand[r, pl.ds(gc, L_)] = ninf16
            thr[g] = ninf16
            cnts[g] = zero16
            return carry

        lax.fori_loop(0, NG_, init_group, 0)

        pltpu.make_async_copy(
            in_hbm.at[b, pl.ds(0, CH_), pl.ds(cbase, SB_)],
            chunks.at[0],
            sems.at[0],
        ).start()

        def run_chunk(ch, carry):
            slot = lax.rem(ch, 2)
            pltpu.make_async_copy(
                in_hbm.at[b, pl.ds(ch * CH_, CH_), pl.ds(cbase, SB_)],
                chunks.at[slot],
                sems.at[slot],
            ).wait()

            @pl.when(ch + 1 < NCHUNK_)
            def _():
                nslot = lax.rem(ch + 1, 2)
                pltpu.make_async_copy(
                    in_hbm.at[b, pl.ds((ch + 1) * CH_, CH_), pl.ds(cbase, SB_)],
                    chunks.at[nslot],
                    sems.at[nslot],
                ).start()

            def run_group(g, carry1):
                gc = g * L_
                cols = lanes + gc

                def run_win(w, carry2):
                    t2, cnt2 = carry2
                    t2, cnt2 = lax.cond(
                        jnp.any(cnt2 > CAP_ - WIN_), fold_all, passthru,
                        g, t2, cnt2,
                    )
                    base = w * WIN_

                    @plsc.parallel_loop(0, WIN_, 8, carry=cnt2)
                    def hot(r, cnt3):
                        for u in range(8):
                            x = chunks[slot, base + r + u, pl.ds(gc, L_)]
                            m = x > t2
                            plsc.store_scatter(cand, [cnt3, cols], x, mask=m)
                            cnt3 = cnt3 + m.astype(jnp.int32)
                        return cnt3

                    return (t2, hot)

                t, cnt = lax.fori_loop(
                    0, CH_ // WIN_, run_win, (thr[g], cnts[g])
                )
                thr[g] = t
                cnts[g] = cnt
                return carry1

            lax.fori_loop(0, NG_, run_group, 0)
            return carry

        lax.fori_loop(0, NCHUNK_, run_chunk, 0)

        def finish_group(g, carry):
            fold_all(g, thr[g], cnts[g])
            return carry

        lax.fori_loop(0, NG_, finish_group, 0)
        pltpu.sync_copy(top, out_hbm.at[b, pl.ds(0, K_TOP_), pl.ds(cbase, SB_)])

    return sc_topk


_SC_TOPK = _make_kernel()


@jax.jit
def kernel(inputs):
    return _SC_TOPK(inputs)


# speedup vs baseline: 46.6923x; 2.9134x over previous
import functools

import jax
import jax.numpy as jnp
from jax import lax
from jax.experimental import pallas as pl
from jax.experimental.pallas import tpu as pltpu
from jax.experimental.pallas import tpu_sc as plsc

K_TOP_ = 64
B_ = 4
S_ = 8192
C_ = 1024
L_ = 16
NW_ = 32
SB_ = 128
NSB_ = C_ // SB_
NG_ = SB_ // L_
CH_ = 256
NCHUNK_ = S_ // CH_
WIN_ = 64
NSLAB_ = 3
CAP_ = 32 * NSLAB_
NEG_ = float("-inf")


def _sort32_asc(v):
    n = 32
    k = 2
    while k <= n:
        j = k // 2
        while j >= 1:
            for i in range(n):
                ix = i ^ j
                if ix > i:
                    a, b = v[i], v[ix]
                    lo = jnp.minimum(a, b)
                    hi = jnp.maximum(a, b)
                    if (i & k) == 0:
                        v[i], v[ix] = lo, hi
                    else:
                        v[i], v[ix] = hi, lo
            j //= 2
        k *= 2


def _bmerge32_desc(v):
    for d in (16, 8, 4, 2, 1):
        for i in range(32):
            if (i % (2 * d)) < d:
                a, b = v[i], v[i + d]
                v[i] = jnp.maximum(a, b)
                v[i + d] = jnp.minimum(a, b)


def _make_kernel():
    mesh = plsc.VectorSubcoreMesh(core_axis_name="c", subcore_axis_name="s")

    @functools.partial(
        pl.kernel,
        mesh=mesh,
        compiler_params=pltpu.CompilerParams(needs_layout_passes=False),
        out_type=jax.ShapeDtypeStruct((B_, K_TOP_, C_), jnp.float32),
        scratch_types=[
            pltpu.VMEM((2, CH_, SB_), jnp.float32),
            pltpu.VMEM((K_TOP_, SB_), jnp.float32),
            pltpu.VMEM((CAP_, SB_), jnp.float32),
            pltpu.VMEM((NG_, L_), jnp.float32),
            pltpu.VMEM((NG_, L_), jnp.int32),
            pltpu.SemaphoreType.DMA((2,)),
        ],
    )
    def sc_topk(in_hbm, out_hbm, chunks, top, cand, thr, cnts, sems):
        wid = lax.axis_index("s") * 2 + lax.axis_index("c")
        b = wid // NSB_
        cbase = (wid % NSB_) * SB_
        lanes = lax.iota(jnp.int32, 16)
        ninf16 = jnp.full((L_,), NEG_, jnp.float32)
        zero16 = jnp.zeros((L_,), jnp.int32)

        def fold(g, s, t, cnt):
            gc = g * L_
            sb = s * 32
            c = [cand[sb + i, pl.ds(gc, L_)] for i in range(32)]
            _sort32_asc(c)
            for jj in range(32):
                top[32 + jj, pl.ds(gc, L_)] = jnp.maximum(
                    top[32 + jj, pl.ds(gc, L_)], c[jj]
                )
            up = [None] * 32
            lo = [None] * 32
            for i in range(32):
                a = top[i, pl.ds(gc, L_)]
                bb = top[32 + i, pl.ds(gc, L_)]
                up[i] = jnp.maximum(a, bb)
                lo[i] = jnp.minimum(a, bb)
            _bmerge32_desc(up)
            for i in range(32):
                top[i, pl.ds(gc, L_)] = up[i]
            _bmerge32_desc(lo)
            for i in range(32):
                top[32 + i, pl.ds(gc, L_)] = lo[i]
                cand[sb + i, pl.ds(gc, L_)] = ninf16
            return top[63, pl.ds(gc, L_)], zero16

        def fold_all(g, t, cnt):
            mx = jnp.max(cnt)
            nslab = (mx + 31) // 32

            def one(s, tc):
                return fold(g, s, *tc)

            return lax.fori_loop(0, nslab, one, (t, cnt))

        def passthru(g, t, cnt):
            return t, cnt

        def init_group(g, carry):
            gc = g * L_
            for r in range(K_TOP_):
                top[r, pl.ds(gc, L_)] = ninf16
            for r in range(CAP_):
                cand[r, pl.ds(gc, L_)] = ninf16
            thr[g] = ninf16
            cnts[g] = zero16
            return carry

        lax.fori_loop(0, NG_, init_group, 0)

        pltpu.make_async_copy(
            in_hbm.at[b, pl.ds(0, CH_), pl.ds(cbase, SB_)],
            chunks.at[0],
            sems.at[0],
        ).start()

        def run_chunk(ch, carry):
            slot = lax.rem(ch, 2)
            pltpu.make_async_copy(
                in_hbm.at[b, pl.ds(ch * CH_, CH_), pl.ds(cbase, SB_)],
                chunks.at[slot],
                sems.at[slot],
            ).wait()

            @pl.when(ch + 1 < NCHUNK_)
            def _():
                nslot = lax.rem(ch + 1, 2)
                pltpu.make_async_copy(
                    in_hbm.at[b, pl.ds((ch + 1) * CH_, CH_), pl.ds(cbase, SB_)],
                    chunks.at[nslot],
                    sems.at[nslot],
                ).start()

            def run_group(g, carry1):
                gc = g * L_
                cols = lanes + gc

                def run_win(w, carry2):
                    t2, cnt2 = carry2
                    t2, cnt2 = lax.cond(
                        jnp.any(cnt2 > CAP_ - WIN_), fold_all, passthru,
                        g, t2, cnt2,
                    )
                    base = w * WIN_

                    @plsc.parallel_loop(0, WIN_, 8, carry=cnt2)
                    def hot(r, cnt3):
                        xs = [
                            chunks[slot, base + r + u, pl.ds(gc, L_)]
                            for u in range(8)
                        ]
                        ms = [x > t2 for x in xs]
                        idx = [cnt3]
                        for u in range(7):
                            idx.append(idx[u] + ms[u].astype(jnp.int32))
                        for u in range(8):
                            plsc.store_scatter(
                                cand, [idx[u], cols], xs[u], mask=ms[u]
                            )
                        return idx[7] + ms[7].astype(jnp.int32)

                    return (t2, hot)

                t, cnt = lax.fori_loop(
                    0, CH_ // WIN_, run_win, (thr[g], cnts[g])
                )
                thr[g] = t
                cnts[g] = cnt
                return carry1

            lax.fori_loop(0, NG_, run_group, 0)
            return carry

        lax.fori_loop(0, NCHUNK_, run_chunk, 0)

        def finish_group(g, carry):
            fold_all(g, thr[g], cnts[g])
            return carry

        lax.fori_loop(0, NG_, finish_group, 0)
        pltpu.sync_copy(top, out_hbm.at[b, pl.ds(0, K_TOP_), pl.ds(cbase, SB_)])

    return sc_topk


_SC_TOPK = _make_kernel()


@jax.jit
def kernel(inputs):
    return _SC_TOPK(inputs)


# speedup vs baseline: 55.1876x; 1.1819x over previous
import functools

import jax
import jax.numpy as jnp
from jax import lax
from jax.experimental import pallas as pl
from jax.experimental.pallas import tpu as pltpu
from jax.experimental.pallas import tpu_sc as plsc

K_TOP_ = 64
B_ = 4
S_ = 8192
C_ = 1024
L_ = 16
NW_ = 32
SB_ = 128
NSB_ = C_ // SB_
NG_ = SB_ // L_
CH_ = 256
NCHUNK_ = S_ // CH_
WIN_ = 128
NSLAB_ = 5
CAP_ = 32 * NSLAB_
NEG_ = float("-inf")


def _sort32_asc(v):
    n = 32
    k = 2
    while k <= n:
        j = k // 2
        while j >= 1:
            for i in range(n):
                ix = i ^ j
                if ix > i:
                    a, b = v[i], v[ix]
                    lo = jnp.minimum(a, b)
                    hi = jnp.maximum(a, b)
                    if (i & k) == 0:
                        v[i], v[ix] = lo, hi
                    else:
                        v[i], v[ix] = hi, lo
            j //= 2
        k *= 2


def _bmerge32_desc(v):
    for d in (16, 8, 4, 2, 1):
        for i in range(32):
            if (i % (2 * d)) < d:
                a, b = v[i], v[i + d]
                v[i] = jnp.maximum(a, b)
                v[i + d] = jnp.minimum(a, b)


def _make_kernel():
    mesh = plsc.VectorSubcoreMesh(core_axis_name="c", subcore_axis_name="s")

    @functools.partial(
        pl.kernel,
        mesh=mesh,
        compiler_params=pltpu.CompilerParams(needs_layout_passes=False),
        out_type=jax.ShapeDtypeStruct((B_, K_TOP_, C_), jnp.float32),
        scratch_types=[
            pltpu.VMEM((2, CH_, SB_), jnp.float32),
            pltpu.VMEM((K_TOP_, SB_), jnp.float32),
            pltpu.VMEM((CAP_ * SB_,), jnp.float32),
            pltpu.VMEM((NG_, L_), jnp.float32),
            pltpu.VMEM((NG_, L_), jnp.int32),
            pltpu.SemaphoreType.DMA((2,)),
        ],
    )
    def sc_topk(in_hbm, out_hbm, chunks, top, cand, thr, cnts, sems):
        wid = lax.axis_index("s") * 2 + lax.axis_index("c")
        b = wid // NSB_
        cbase = (wid % NSB_) * SB_
        lanes = lax.iota(jnp.int32, 16)
        ninf16 = jnp.full((L_,), NEG_, jnp.float32)
        zero16 = jnp.zeros((L_,), jnp.int32)

        def fold(g, s, t, cnt):
            gc = g * L_
            sb = s * 32
            c = [cand[pl.ds((sb + i) * SB_ + gc, L_)] for i in range(32)]
            _sort32_asc(c)
            for jj in range(32):
                top[32 + jj, pl.ds(gc, L_)] = jnp.maximum(
                    top[32 + jj, pl.ds(gc, L_)], c[jj]
                )
            up = [None] * 32
            lo = [None] * 32
            for i in range(32):
                a = top[i, pl.ds(gc, L_)]
                bb = top[32 + i, pl.ds(gc, L_)]
                up[i] = jnp.maximum(a, bb)
                lo[i] = jnp.minimum(a, bb)
            _bmerge32_desc(up)
            for i in range(32):
                top[i, pl.ds(gc, L_)] = up[i]
            _bmerge32_desc(lo)
            for i in range(32):
                top[32 + i, pl.ds(gc, L_)] = lo[i]
                cand[pl.ds((sb + i) * SB_ + gc, L_)] = ninf16
            return top[63, pl.ds(gc, L_)], zero16

        def fold_all(g, t, addr):
            cols = lanes + g * L_
            cnt = lax.shift_right_logical(addr - cols, 7)
            mx = jnp.max(cnt)
            nslab = (mx + 31) // 32

            def one(s, tc):
                t1, _ = fold(g, s, tc[0], tc[1])
                return t1, cols

            return lax.fori_loop(0, nslab, one, (t, cols))

        def passthru(g, t, addr):
            return t, addr

        def init_group(g, carry):
            gc = g * L_
            for r in range(K_TOP_):
                top[r, pl.ds(gc, L_)] = ninf16
            for r in range(CAP_):
                cand[pl.ds(r * SB_ + gc, L_)] = ninf16
            thr[g] = ninf16
            cnts[g] = lanes + gc
            return carry

        lax.fori_loop(0, NG_, init_group, 0)

        pltpu.make_async_copy(
            in_hbm.at[b, pl.ds(0, CH_), pl.ds(cbase, SB_)],
            chunks.at[0],
            sems.at[0],
        ).start()

        def run_chunk(ch, carry):
            slot = lax.rem(ch, 2)
            pltpu.make_async_copy(
                in_hbm.at[b, pl.ds(ch * CH_, CH_), pl.ds(cbase, SB_)],
                chunks.at[slot],
                sems.at[slot],
            ).wait()

            @pl.when(ch + 1 < NCHUNK_)
            def _():
                nslot = lax.rem(ch + 1, 2)
                pltpu.make_async_copy(
                    in_hbm.at[b, pl.ds((ch + 1) * CH_, CH_), pl.ds(cbase, SB_)],
                    chunks.at[nslot],
                    sems.at[nslot],
                ).start()

            def run_group(g, carry1):
                gc = g * L_
                cols = lanes + gc

                def run_win(w, carry2):
                    t2, addr2 = carry2
                    t2, addr2 = lax.cond(
                        jnp.any(addr2 - cols > (CAP_ - WIN_) * SB_),
                        fold_all, passthru, g, t2, addr2,
                    )
                    base = w * WIN_

                    @plsc.parallel_loop(0, WIN_, 8, carry=addr2)
                    def hot(r, addr3):
                        xs = [
                            chunks[slot, base + r + u, pl.ds(gc, L_)]
                            for u in range(8)
                        ]
                        ms = [x > t2 for x in xs]
                        stp = [
                            jnp.where(m, jnp.int32(SB_), jnp.int32(0))
                            for m in ms
                        ]
                        idx = [addr3]
                        for u in range(7):
                            idx.append(idx[u] + stp[u])
                        for u in range(8):
                            plsc.store_scatter(
                                cand, [idx[u]], xs[u], mask=ms[u]
                            )
                        return idx[7] + stp[7]

                    return (t2, hot)

                t, cnt = lax.fori_loop(
                    0, CH_ // WIN_, run_win, (thr[g], cnts[g])
                )
                thr[g] = t
                cnts[g] = cnt
                return carry1

            lax.fori_loop(0, NG_, run_group, 0)
            return carry

        lax.fori_loop(0, NCHUNK_, run_chunk, 0)

        def finish_group(g, carry):
            fold_all(g, thr[g], cnts[g])
            return carry

        lax.fori_loop(0, NG_, finish_group, 0)
        pltpu.sync_copy(top, out_hbm.at[b, pl.ds(0, K_TOP_), pl.ds(cbase, SB_)])

    return sc_topk


_SC_TOPK = _make_kernel()


@jax.jit
def kernel(inputs):
    return _SC_TOPK(inputs)


# speedup vs baseline: 59.6312x; 1.0805x over previous
import functools

import jax
import jax.numpy as jnp
from jax import lax
from jax.experimental import pallas as pl
from jax.experimental.pallas import tpu as pltpu
from jax.experimental.pallas import tpu_sc as plsc

K_TOP_ = 64
B_ = 4
S_ = 8192
C_ = 1024
L_ = 16
NW_ = 32
SB_ = 128
NSB_ = C_ // SB_
NG_ = SB_ // L_
CH_ = 256
NCHUNK_ = S_ // CH_
WIN_ = 256
NSLAB_ = 9
CAP_ = 32 * NSLAB_
NEG_ = float("-inf")


def _sort32_asc(v):
    n = 32
    k = 2
    while k <= n:
        j = k // 2
        while j >= 1:
            for i in range(n):
                ix = i ^ j
                if ix > i:
                    a, b = v[i], v[ix]
                    lo = jnp.minimum(a, b)
                    hi = jnp.maximum(a, b)
                    if (i & k) == 0:
                        v[i], v[ix] = lo, hi
                    else:
                        v[i], v[ix] = hi, lo
            j //= 2
        k *= 2


def _bmerge32_desc(v):
    for d in (16, 8, 4, 2, 1):
        for i in range(32):
            if (i % (2 * d)) < d:
                a, b = v[i], v[i + d]
                v[i] = jnp.maximum(a, b)
                v[i + d] = jnp.minimum(a, b)


def _make_kernel():
    mesh = plsc.VectorSubcoreMesh(core_axis_name="c", subcore_axis_name="s")

    @functools.partial(
        pl.kernel,
        mesh=mesh,
        compiler_params=pltpu.CompilerParams(needs_layout_passes=False),
        out_type=jax.ShapeDtypeStruct((B_, K_TOP_, C_), jnp.float32),
        scratch_types=[
            pltpu.VMEM((2, CH_, SB_), jnp.float32),
            pltpu.VMEM((K_TOP_, SB_), jnp.float32),
            pltpu.VMEM((CAP_ * SB_,), jnp.float32),
            pltpu.VMEM((NG_, L_), jnp.float32),
            pltpu.VMEM((NG_, L_), jnp.int32),
            pltpu.SemaphoreType.DMA((2,)),
        ],
    )
    def sc_topk(in_hbm, out_hbm, chunks, top, cand, thr, cnts, sems):
        wid = lax.axis_index("s") * 2 + lax.axis_index("c")
        b = wid // NSB_
        cbase = (wid % NSB_) * SB_
        lanes = lax.iota(jnp.int32, 16)
        ninf16 = jnp.full((L_,), NEG_, jnp.float32)
        zero16 = jnp.zeros((L_,), jnp.int32)

        def fold(g, s, t, cnt):
            gc = g * L_
            sb = s * 32
            c = [cand[pl.ds((sb + i) * SB_ + gc, L_)] for i in range(32)]
            _sort32_asc(c)
            for jj in range(32):
                top[32 + jj, pl.ds(gc, L_)] = jnp.maximum(
                    top[32 + jj, pl.ds(gc, L_)], c[jj]
                )
            up = [None] * 32
            lo = [None] * 32
            for i in range(32):
                a = top[i, pl.ds(gc, L_)]
                bb = top[32 + i, pl.ds(gc, L_)]
                up[i] = jnp.maximum(a, bb)
                lo[i] = jnp.minimum(a, bb)
            _bmerge32_desc(up)
            for i in range(32):
                top[i, pl.ds(gc, L_)] = up[i]
            _bmerge32_desc(lo)
            for i in range(32):
                top[32 + i, pl.ds(gc, L_)] = lo[i]
                cand[pl.ds((sb + i) * SB_ + gc, L_)] = ninf16
            return top[63, pl.ds(gc, L_)], zero16

        def fold_all(g, t, addr):
            cols = lanes + g * L_
            cnt = lax.shift_right_logical(addr - cols, 7)
            mx = jnp.max(cnt)
            nslab = (mx + 31) // 32

            def one(s, tc):
                t1, _ = fold(g, s, tc[0], tc[1])
                return t1, cols

            return lax.fori_loop(0, nslab, one, (t, cols))

        def passthru(g, t, addr):
            return t, addr

        def init_group(g, carry):
            gc = g * L_
            for r in range(K_TOP_):
                top[r, pl.ds(gc, L_)] = ninf16
            for r in range(CAP_):
                cand[pl.ds(r * SB_ + gc, L_)] = ninf16
            thr[g] = ninf16
            cnts[g] = lanes + gc
            return carry

        lax.fori_loop(0, NG_, init_group, 0)

        pltpu.make_async_copy(
            in_hbm.at[b, pl.ds(0, CH_), pl.ds(cbase, SB_)],
            chunks.at[0],
            sems.at[0],
        ).start()

        def run_chunk(ch, carry):
            slot = lax.rem(ch, 2)
            pltpu.make_async_copy(
                in_hbm.at[b, pl.ds(ch * CH_, CH_), pl.ds(cbase, SB_)],
                chunks.at[slot],
                sems.at[slot],
            ).wait()

            @pl.when(ch + 1 < NCHUNK_)
            def _():
                nslot = lax.rem(ch + 1, 2)
                pltpu.make_async_copy(
                    in_hbm.at[b, pl.ds((ch + 1) * CH_, CH_), pl.ds(cbase, SB_)],
                    chunks.at[nslot],
                    sems.at[nslot],
                ).start()

            def run_group(g, carry1):
                gc = g * L_
                cols = lanes + gc

                def run_win(w, carry2):
                    t2, addr2 = carry2
                    del w
                    t2, addr2 = lax.cond(
                        jnp.any(addr2 - cols > (CAP_ - WIN_) * SB_),
                        fold_all, passthru, g, t2, addr2,
                    )
                    base = 0

                    @plsc.parallel_loop(0, WIN_, 8, carry=addr2)
                    def hot(r, addr3):
                        xs = [
                            chunks[slot, base + r + u, pl.ds(gc, L_)]
                            for u in range(8)
                        ]
                        ms = [x > t2 for x in xs]
                        stp = [
                            jnp.where(m, jnp.int32(SB_), jnp.int32(0))
                            for m in ms
                        ]
                        idx = [addr3]
                        for u in range(7):
                            idx.append(idx[u] + stp[u])
                        for u in range(8):
                            plsc.store_scatter(
                                cand, [idx[u]], xs[u], mask=ms[u]
                            )
                        return idx[7] + stp[7]

                    return (t2, hot)

                t, cnt = lax.fori_loop(
                    0, CH_ // WIN_, run_win, (thr[g], cnts[g])
                )
                thr[g] = t
                cnts[g] = cnt
                return carry1

            lax.fori_loop(0, NG_, run_group, 0)
            return carry

        lax.fori_loop(0, NCHUNK_, run_chunk, 0)

        def finish_group(g, carry):
            fold_all(g, thr[g], cnts[g])
            return carry

        lax.fori_loop(0, NG_, finish_group, 0)
        pltpu.sync_copy(top, out_hbm.at[b, pl.ds(0, K_TOP_), pl.ds(cbase, SB_)])

    return sc_topk


_SC_TOPK = _make_kernel()


@jax.jit
def kernel(inputs):
    return _SC_TOPK(inputs)


# speedup vs baseline: 65.1364x; 1.0923x over previous
import functools

import jax
import jax.numpy as jnp
from jax import lax
from jax.experimental import pallas as pl
from jax.experimental.pallas import tpu as pltpu
from jax.experimental.pallas import tpu_sc as plsc

K_TOP_ = 64
B_ = 4
S_ = 8192
C_ = 1024
L_ = 16
NW_ = 32
SB_ = 128
NSB_ = C_ // SB_
NG_ = SB_ // L_
CH_ = 256
NCHUNK_ = S_ // CH_
WIN_ = 256
NSLAB_ = 9
CAP_ = 32 * NSLAB_
NEG_ = float("-inf")


def _sort32_asc(v):
    n = 32
    k = 2
    while k <= n:
        j = k // 2
        while j >= 1:
            for i in range(n):
                ix = i ^ j
                if ix > i:
                    a, b = v[i], v[ix]
                    lo = jnp.minimum(a, b)
                    hi = jnp.maximum(a, b)
                    if (i & k) == 0:
                        v[i], v[ix] = lo, hi
                    else:
                        v[i], v[ix] = hi, lo
            j //= 2
        k *= 2


def _bmerge32_desc(v):
    for d in (16, 8, 4, 2, 1):
        for i in range(32):
            if (i % (2 * d)) < d:
                a, b = v[i], v[i + d]
                v[i] = jnp.maximum(a, b)
                v[i + d] = jnp.minimum(a, b)


def _make_kernel():
    mesh = plsc.VectorSubcoreMesh(core_axis_name="c", subcore_axis_name="s")

    @functools.partial(
        pl.kernel,
        mesh=mesh,
        compiler_params=pltpu.CompilerParams(needs_layout_passes=False),
        out_type=jax.ShapeDtypeStruct((B_, K_TOP_, C_), jnp.float32),
        scratch_types=[
            pltpu.VMEM((2, CH_, SB_), jnp.float32),
            pltpu.VMEM((K_TOP_, SB_), jnp.float32),
            pltpu.VMEM((CAP_ * SB_,), jnp.float32),
            pltpu.VMEM((NG_, L_), jnp.float32),
            pltpu.VMEM((NG_, L_), jnp.int32),
            pltpu.SemaphoreType.DMA((2,)),
        ],
    )
    def sc_topk(in_hbm, out_hbm, chunks, top, cand, thr, cnts, sems):
        wid = lax.axis_index("s") * 2 + lax.axis_index("c")
        b = wid // NSB_
        cbase = (wid % NSB_) * SB_
        lanes = lax.iota(jnp.int32, 16)
        ninf16 = jnp.full((L_,), NEG_, jnp.float32)
        zero16 = jnp.zeros((L_,), jnp.int32)

        def fold(g, s, t, cnt):
            gc = g * L_
            sb = s * 32
            c = [cand[pl.ds((sb + i) * SB_ + gc, L_)] for i in range(32)]
            _sort32_asc(c)
            for jj in range(32):
                top[32 + jj, pl.ds(gc, L_)] = jnp.maximum(
                    top[32 + jj, pl.ds(gc, L_)], c[jj]
                )
            up = [None] * 32
            lo = [None] * 32
            for i in range(32):
                a = top[i, pl.ds(gc, L_)]
                bb = top[32 + i, pl.ds(gc, L_)]
                up[i] = jnp.maximum(a, bb)
                lo[i] = jnp.minimum(a, bb)
            _bmerge32_desc(up)
            for i in range(32):
                top[i, pl.ds(gc, L_)] = up[i]
            _bmerge32_desc(lo)
            for i in range(32):
                top[32 + i, pl.ds(gc, L_)] = lo[i]
                cand[pl.ds((sb + i) * SB_ + gc, L_)] = ninf16
            return top[63, pl.ds(gc, L_)], zero16

        def fold_all(g, t, addr):
            cnt = lax.shift_right_logical(addr, 7)
            mx = jnp.max(cnt)
            nslab = (mx + 31) // 32

            def one(s, tc):
                t1, _ = fold(g, s, tc[0], tc[1])
                return t1, zero16

            return lax.fori_loop(0, nslab, one, (t, zero16))

        def passthru(g, t, addr):
            return t, addr

        def init_group(g, carry):
            gc = g * L_
            for r in range(K_TOP_):
                top[r, pl.ds(gc, L_)] = ninf16
            for r in range(CAP_):
                cand[pl.ds(r * SB_ + gc, L_)] = ninf16
            thr[g] = ninf16
            cnts[g] = zero16
            return carry

        lax.fori_loop(0, NG_, init_group, 0)

        pltpu.make_async_copy(
            in_hbm.at[b, pl.ds(0, CH_), pl.ds(cbase, SB_)],
            chunks.at[0],
            sems.at[0],
        ).start()

        def run_chunk(ch, carry):
            slot = lax.rem(ch, 2)
            pltpu.make_async_copy(
                in_hbm.at[b, pl.ds(ch * CH_, CH_), pl.ds(cbase, SB_)],
                chunks.at[slot],
                sems.at[slot],
            ).wait()

            @pl.when(ch + 1 < NCHUNK_)
            def _():
                nslot = lax.rem(ch + 1, 2)
                pltpu.make_async_copy(
                    in_hbm.at[b, pl.ds((ch + 1) * CH_, CH_), pl.ds(cbase, SB_)],
                    chunks.at[nslot],
                    sems.at[nslot],
                ).start()

            def run_group(g, carry1):
                gc = g * L_
                cols = lanes + gc

                def run_win(w, carry2):
                    t2, addr2 = carry2
                    del w
                    t2, addr2 = lax.cond(
                        jnp.any(addr2 > (CAP_ - WIN_) * SB_),
                        fold_all, passthru, g, t2, addr2,
                    )
                    base = 0

                    @plsc.parallel_loop(0, WIN_, 8, carry=addr2)
                    def hot(r, addr3):
                        xs = [
                            chunks[slot, base + r + u, pl.ds(gc, L_)]
                            for u in range(8)
                        ]
                        ms = [x > t2 for x in xs]
                        stp = [
                            jnp.where(m, jnp.int32(SB_), jnp.int32(0))
                            for m in ms
                        ]
                        off = [addr3]
                        for u in range(7):
                            off.append(off[u] + stp[u])
                        for u in range(8):
                            plsc.store_scatter(
                                cand, [off[u] + cols], xs[u], mask=ms[u]
                            )
                        return off[7] + stp[7]

                    return (t2, hot)

                t, cnt = lax.fori_loop(
                    0, CH_ // WIN_, run_win, (thr[g], cnts[g])
                )
                thr[g] = t
                cnts[g] = cnt
                return carry1

            lax.fori_loop(0, NG_, run_group, 0)
            return carry

        lax.fori_loop(0, NCHUNK_, run_chunk, 0)

        def finish_group(g, carry):
            fold_all(g, thr[g], cnts[g])
            return carry

        lax.fori_loop(0, NG_, finish_group, 0)
        pltpu.sync_copy(top, out_hbm.at[b, pl.ds(0, K_TOP_), pl.ds(cbase, SB_)])

    return sc_topk


_SC_TOPK = _make_kernel()


@jax.jit
def kernel(inputs):
    return _SC_TOPK(inputs)
